# baseline (speedup 1.0000x reference)
import jax
import jax.numpy as jnp
from jax.experimental import pallas as pl
from jax.experimental.pallas import tpu as pltpu

_EPS = 0.1


def _zono_body(x_ref, o_ref):
    pv = x_ref[0, 0]
    low = pv < _EPS
    high = pv > 1.0 - _EPS
    new_pv = jnp.where(low, (pv + _EPS) * 0.5,
             jnp.where(high, (pv + 1.0 - _EPS) * 0.5, pv))
    new_e = jnp.where(low, (_EPS + pv) * 0.5,
            jnp.where(high, (1.0 - pv + _EPS) * 0.5, jnp.full_like(pv, _EPS)))
    _, c, h, w = o_ref.shape
    ci = jax.lax.broadcasted_iota(jnp.int32, (c, h, w), 0)
    hi = jax.lax.broadcasted_iota(jnp.int32, (c, h, w), 1)
    wi = jax.lax.broadcasted_iota(jnp.int32, (c, h, w), 2)
    diag = ci == 1 + hi * w + wi
    out = jnp.where(ci == 0, new_pv, jnp.where(diag, new_e, 0.0))
    o_ref[0] = out


def kernel(x):
    B, _, H, W = x.shape
    HW = H * W
    return pl.pallas_call(
        _zono_body,
        grid=(B,),
        in_specs=[pl.BlockSpec((1, 1, H, W), lambda b: (b, 0, 0, 0))],
        out_specs=pl.BlockSpec((1, 1 + HW, H, W), lambda b: (b, 0, 0, 0)),
        out_shape=jax.ShapeDtypeStruct((B, 1 + HW, H, W), x.dtype),
        compiler_params=pltpu.CompilerParams(
            dimension_semantics=("parallel",),
        ),
    )(x)


# speedup vs baseline: 3.4513x; 3.4513x over previous
import jax
import jax.numpy as jnp
from jax.experimental import pallas as pl
from jax.experimental.pallas import tpu as pltpu

_EPS = 0.1


def _zono_body(x_ref, o_ref):
    pv = x_ref[0]
    low = pv < _EPS
    high = pv > 1.0 - _EPS
    new_pv = jnp.where(low, (pv + _EPS) * 0.5,
             jnp.where(high, (pv + 1.0 - _EPS) * 0.5, pv))
    new_e = jnp.where(low, (_EPS + pv) * 0.5,
            jnp.where(high, (1.0 - pv + _EPS) * 0.5, jnp.full_like(pv, _EPS)))
    c, hw = o_ref.shape[1], o_ref.shape[2]
    row = jax.lax.broadcasted_iota(jnp.int32, (c, hw), 0)
    col = jax.lax.broadcasted_iota(jnp.int32, (c, hw), 1)
    out = jnp.where(row == 0, new_pv, jnp.where(row == col + 1, new_e, 0.0))
    o_ref[0] = out


def kernel(x):
    B, _, H, W = x.shape
    HW = H * W
    xf = x.reshape(B, 1, HW)
    out = pl.pallas_call(
        _zono_body,
        grid=(B,),
        in_specs=[pl.BlockSpec((1, 1, HW), lambda b: (b, 0, 0))],
        out_specs=pl.BlockSpec((1, 1 + HW, HW), lambda b: (b, 0, 0)),
        out_shape=jax.ShapeDtypeStruct((B, 1 + HW, HW), x.dtype),
        compiler_params=pltpu.CompilerParams(
            dimension_semantics=("parallel",),
        ),
    )(xf)
    return out.reshape(B, 1 + HW, H, W)


# speedup vs baseline: 3.5337x; 1.0239x over previous
import jax
import jax.numpy as jnp
from jax.experimental import pallas as pl
from jax.experimental.pallas import tpu as pltpu

_EPS = 0.1


def _zono_body(x_ref, o_ref):
    pv = x_ref[:, 0]
    low = pv < _EPS
    high = pv > 1.0 - _EPS
    new_pv = jnp.where(low, (pv + _EPS) * 0.5,
             jnp.where(high, (pv + 1.0 - _EPS) * 0.5, pv))
    new_e = jnp.where(low, (_EPS + pv) * 0.5,
            jnp.where(high, (1.0 - pv + _EPS) * 0.5, jnp.full_like(pv, _EPS)))
    bb, c, hw = o_ref.shape
    row = jax.lax.broadcasted_iota(jnp.int32, (c, hw), 0)
    col = jax.lax.broadcasted_iota(jnp.int32, (c, hw), 1)
    diag = row == col + 1
    ctr = row == 0
    for i in range(bb):
        o_ref[i] = jnp.where(ctr, new_pv[i:i + 1],
                   jnp.where(diag, new_e[i:i + 1], 0.0))


def kernel(x):
    B, _, H, W = x.shape
    HW = H * W
    xf = x.reshape(B, 1, HW)
    BB = 4
    out = pl.pallas_call(
        _zono_body,
        grid=(B // BB,),
        in_specs=[pl.BlockSpec((BB, 1, HW), lambda b: (b, 0, 0))],
        out_specs=pl.BlockSpec((BB, 1 + HW, HW), lambda b: (b, 0, 0)),
        out_shape=jax.ShapeDtypeStruct((B, 1 + HW, HW), x.dtype),
        compiler_params=pltpu.CompilerParams(
            dimension_semantics=("arbitrary",),
        ),
    )(xf)
    return out.reshape(B, 1 + HW, H, W)


# speedup vs baseline: 14.7591x; 4.1766x over previous
import jax
import jax.numpy as jnp
from jax.experimental import pallas as pl
from jax.experimental.pallas import tpu as pltpu

_EPS = 0.1


def _zono_body(x_ref, o_ref):
    h = pl.program_id(0)
    _, w_blk, c, b = o_ref.shape
    pv = x_ref[0]
    low = pv < _EPS
    high = pv > 1.0 - _EPS
    new_pv = jnp.where(low, (pv + _EPS) * 0.5,
             jnp.where(high, (pv + 1.0 - _EPS) * 0.5, pv))
    new_e = jnp.where(low, (_EPS + pv) * 0.5,
            jnp.where(high, (1.0 - pv + _EPS) * 0.5, jnp.full_like(pv, _EPS)))
    o_ref[0] = jnp.zeros((w_blk, c, b), o_ref.dtype)
    for w in range(w_blk):
        o_ref[0, w, 0:1, :] = new_pv[w]
        k = 1 + h * w_blk + w
        o_ref[0, w, pl.ds(k, 1), :] = new_e[w]


def kernel(x):
    B, _, H, W = x.shape
    C = 1 + H * W
    xt = jnp.transpose(x, (2, 3, 1, 0))
    out = pl.pallas_call(
        _zono_body,
        grid=(H,),
        in_specs=[pl.BlockSpec((1, W, 1, B), lambda h: (h, 0, 0, 0))],
        out_specs=pl.BlockSpec((1, W, C, B), lambda h: (h, 0, 0, 0)),
        out_shape=jax.ShapeDtypeStruct((H, W, C, B), x.dtype),
        compiler_params=pltpu.CompilerParams(
            dimension_semantics=("arbitrary",),
        ),
    )(xt)
    return jnp.transpose(out, (3, 2, 0, 1))
